# Initial kernel scaffold; baseline (speedup 1.0000x reference)
import jax
import jax.numpy as jnp
from jax.experimental import pallas as pl


def kernel(features, edge_index, edge_type, W1, b1, a_src, a_dst, rel_emb, W_lat, W_out, b_out, W2, b2):
    raise NotImplementedError("write your pallas kernel here")


# speedup vs baseline: 4.9448x; 4.9448x over previous
import dataclasses

import jax
import jax.numpy as jnp
from jax import lax
from jax.experimental import pallas as pl
from jax.experimental.pallas import tpu as pltpu
from jax.experimental.pallas import tpu_sc as plsc

N_NODES = 10000
N_EDGES = 320000
X_DIM = 128
BASE_DIM = 128
LATENT_DIM = 32

NC = 2
NS = 16
L = 16
GR = 32
EPS = N_EDGES // NS
NCH = 5
NG = EPS // NCH // GR
HALF = 5120
TRASH = HALF
RPS = HALF // NS

_f32 = jnp.float32
_i32 = jnp.int32


def _tc_front_body(f_ref, w1_ref, b1_ref, a2_ref, h1_ref, pv_ref):
    h1 = jnp.maximum(
        jnp.dot(f_ref[...], w1_ref[...], preferred_element_type=_f32)
        + b1_ref[...],
        0.0,
    )
    h1_ref[...] = h1
    pv_ref[...] = jnp.dot(h1, a2_ref[...], preferred_element_type=_f32)


def _tc_front(features, W1, b1, a2):
    BN = 1000
    grid = (N_NODES // BN,)
    return pl.pallas_call(
        _tc_front_body,
        grid=grid,
        in_specs=[
            pl.BlockSpec((BN, X_DIM), lambda i: (i, 0)),
            pl.BlockSpec((X_DIM, BASE_DIM), lambda i: (0, 0)),
            pl.BlockSpec((1, BASE_DIM), lambda i: (0, 0)),
            pl.BlockSpec((BASE_DIM, 8), lambda i: (0, 0)),
        ],
        out_specs=[
            pl.BlockSpec((BN, BASE_DIM), lambda i: (i, 0)),
            pl.BlockSpec((BN, 8), lambda i: (i, 0)),
        ],
        out_shape=[
            jax.ShapeDtypeStruct((N_NODES, BASE_DIM), _f32),
            jax.ShapeDtypeStruct((N_NODES, 8), _f32),
        ],
    )(features, W1, b1, a2)


def _sc_edge_body(h1_hbm, psrc_hbm, pdst_hbm, rel_hbm, src_hbm, dst_hbm,
                  et_hbm, aggp_hbm,
                  psrc_v, pdst_v, rel_v, src2, dst2, et2, x_v, lidx, grows,
                  srows, agg_sh):
    cid = lax.axis_index("c")
    sid = lax.axis_index("s")
    nlo = cid * HALF

    pltpu.sync_copy(psrc_hbm, psrc_v)
    pltpu.sync_copy(pdst_hbm, pdst_v)
    pltpu.sync_copy(rel_hbm, rel_v)

    zeros16 = jnp.zeros((L,), _f32)

    @pl.loop(0, GR)
    def _zero_rows(r):
        for c in range(0, BASE_DIM, L):
            srows[r, pl.ds(c, L)] = zeros16

    row0 = sid * RPS
    for k in range(RPS // GR):
        pltpu.sync_copy(srows, agg_sh.at[pl.ds(row0 + k * GR, GR), :])

    @pl.when(sid == 0)
    def _zero_trash():
        pltpu.sync_copy(srows.at[pl.ds(0, 8), :],
                        agg_sh.at[pl.ds(HALF, 8), :])

    plsc.subcore_barrier()

    for st in range(NCH):
        plane = sid * NCH + st
        pltpu.sync_copy(src_hbm.at[plane], src2)
        pltpu.sync_copy(dst_hbm.at[plane], dst2)
        pltpu.sync_copy(et_hbm.at[plane], et2)

        @pl.loop(0, NG)
        def _granule(gi):
            for j in range(0, GR, L):
                sv = src2[gi, pl.ds(j, L)]
                dv = dst2[gi, pl.ds(j, L)]
                tv = et2[gi, pl.ds(j, L)]
                e = (plsc.load_gather(psrc_v, [sv])
                     + plsc.load_gather(pdst_v, [dv])
                     + plsc.load_gather(rel_v, [tv]))
                e = jnp.where(e >= 0.0, e, 0.2 * e)
                x_v[pl.ds(j, L)] = jnp.exp(e)
                lv = dv - nlo
                inb = (lv >= 0) & (lv < HALF)
                lidx[pl.ds(j, L)] = jnp.where(inb, lv, TRASH)

            pltpu.sync_copy(h1_hbm.at[src2.at[gi]], grows)

            @pl.loop(0, GR)
            def _scale(r):
                xs = plsc.load_gather(x_v, [jnp.full((L,), r, _i32)])
                for c in range(0, BASE_DIM, L):
                    srows[r, pl.ds(c, L)] = grows[r, pl.ds(c, L)] * xs

            pltpu.sync_copy(srows, agg_sh.at[lidx], add=True)

    plsc.subcore_barrier()

    pltpu.sync_copy(agg_sh.at[pl.ds(row0, RPS)],
                    aggp_hbm.at[cid, pl.ds(row0, RPS), :])


def _sc_edge(h1, p_src, p_dst, rel16, src2, dst2, et2):
    mesh = plsc.VectorSubcoreMesh(core_axis_name="c", subcore_axis_name="s")
    cp = pltpu.CompilerParams()
    if "needs_layout_passes" in pltpu.CompilerParams.__dataclass_fields__:
        cp = dataclasses.replace(cp, needs_layout_passes=False)
    kern = pl.kernel(
        _sc_edge_body,
        out_type=jax.ShapeDtypeStruct((NC, HALF, BASE_DIM), _f32),
        mesh=mesh,
        scratch_types=[
            pltpu.VMEM((N_NODES,), _f32),
            pltpu.VMEM((N_NODES,), _f32),
            pltpu.VMEM((L,), _f32),
            pltpu.VMEM((NG, GR), _i32),
            pltpu.VMEM((NG, GR), _i32),
            pltpu.VMEM((NG, GR), _i32),
            pltpu.VMEM((GR,), _f32),
            pltpu.VMEM((GR,), _i32),
            pltpu.VMEM((GR, BASE_DIM), _f32),
            pltpu.VMEM((GR, BASE_DIM), _f32),
            pltpu.VMEM_SHARED((HALF + 8, BASE_DIM), _f32),
        ],
        compiler_params=cp,
    )
    return kern(h1, p_src, p_dst, rel16, src2, dst2, et2)


NPAD2 = 10240


def _sc_den_body(psrc_hbm, pdst_hbm, rel_hbm, src_hbm, dst_hbm, et_hbm,
                 den_hbm, psrc_v, pdst_v, rel_v, srcd, dstd, etd, x_v, den_v,
                 den_sh):
    cid = lax.axis_index("c")
    sid = lax.axis_index("s")
    row0d = sid * (NPAD2 // NS)

    pltpu.sync_copy(psrc_hbm, psrc_v)
    pltpu.sync_copy(pdst_hbm, pdst_v)
    pltpu.sync_copy(rel_hbm, rel_v)

    zeros16 = jnp.zeros((L,), _f32)
    for j in range(0, GR, L):
        x_v[pl.ds(j, L)] = zeros16
    for k in range((NPAD2 // NS) // GR):
        pltpu.sync_copy(x_v, den_sh.at[pl.ds(row0d + k * GR, GR)])

    plsc.subcore_barrier()

    for st in range(NCH):
        plane = sid * NCH + st
        pltpu.sync_copy(src_hbm.at[plane], srcd)
        pltpu.sync_copy(dst_hbm.at[plane], dstd)
        pltpu.sync_copy(et_hbm.at[plane], etd)

        @pl.loop(0, NG)
        def _granule(gi):
            for j in range(0, GR, L):
                sv = srcd[gi, pl.ds(j, L)]
                dv = dstd[gi, pl.ds(j, L)]
                tv = etd[gi, pl.ds(j, L)]
                e = (plsc.load_gather(psrc_v, [sv])
                     + plsc.load_gather(pdst_v, [dv])
                     + plsc.load_gather(rel_v, [tv]))
                e = jnp.where(e >= 0.0, e, 0.2 * e)
                x_v[pl.ds(j, L)] = jnp.exp(e)
            pltpu.sync_copy(x_v, den_sh.at[dstd.at[gi]], add=True)

    plsc.subcore_barrier()

    pltpu.sync_copy(den_sh.at[pl.ds(cid * HALF + sid * RPS, RPS)], den_v)
    pltpu.sync_copy(den_v, den_hbm.at[cid * NS + sid, 0])


def _sc_den(p_src, p_dst, rel16, src2, dst2, et2):
    mesh = plsc.VectorSubcoreMesh(core_axis_name="c", subcore_axis_name="s")
    cp = pltpu.CompilerParams()
    if "needs_layout_passes" in pltpu.CompilerParams.__dataclass_fields__:
        cp = dataclasses.replace(cp, needs_layout_passes=False)
    kern = pl.kernel(
        _sc_den_body,
        out_type=jax.ShapeDtypeStruct((NC * NS, 1, RPS), _f32),
        mesh=mesh,
        scratch_types=[
            pltpu.VMEM((N_NODES,), _f32),
            pltpu.VMEM((N_NODES,), _f32),
            pltpu.VMEM((L,), _f32),
            pltpu.VMEM((NG, GR), _i32),
            pltpu.VMEM((NG, GR), _i32),
            pltpu.VMEM((NG, GR), _i32),
            pltpu.VMEM((GR,), _f32),
            pltpu.VMEM((RPS,), _f32),
            pltpu.VMEM_SHARED((NPAD2,), _f32),
        ],
        compiler_params=cp,
    )
    return kern(p_src, p_dst, rel16, src2, dst2, et2)


def _tc_back_body(aggp_ref, den_ref, wlat_ref, wout_ref, bout_ref, w2_ref,
                  b2_ref, lat_ref, h3_ref):
    s = jnp.concatenate([aggp_ref[0], aggp_ref[1]], axis=0)
    agg = s / (den_ref[...] + 1e-16)
    lat = jnp.dot(agg, wlat_ref[...], preferred_element_type=_f32)
    lat_ref[...] = lat[:N_NODES]
    h2 = jnp.dot(agg, wout_ref[...], preferred_element_type=_f32) + bout_ref[...]
    h3 = (
        jnp.dot(jnp.maximum(h2, 0.0), w2_ref[...], preferred_element_type=_f32)
        + b2_ref[...]
    )
    h3_ref[...] = h3[:N_NODES]


def _tc_back(aggp, den, W_lat, W_out, b_out, W2, b2):
    return pl.pallas_call(
        _tc_back_body,
        out_shape=[
            jax.ShapeDtypeStruct((N_NODES, LATENT_DIM), _f32),
            jax.ShapeDtypeStruct((N_NODES, X_DIM), _f32),
        ],
    )(aggp, den, W_lat, W_out, b_out, W2, b2)


def kernel(features, edge_index, edge_type, W1, b1, a_src, a_dst, rel_emb,
           W_lat, W_out, b_out, W2, b2):
    a2 = jnp.concatenate(
        [a_src[:, None], a_dst[:, None], jnp.zeros((BASE_DIM, 6), _f32)],
        axis=1,
    )
    h1, pv = _tc_front(features, W1, b1.reshape(1, BASE_DIM), a2)
    p_src = pv[:, 0]
    p_dst = pv[:, 1]

    rel16 = jnp.pad(rel_emb.astype(_f32), (0, L - rel_emb.shape[0]))
    src2 = edge_index[0].reshape(NS * NCH, NG, GR)
    dst2 = edge_index[1].reshape(NS * NCH, NG, GR)
    et2 = edge_type.reshape(NS * NCH, NG, GR)

    aggp = _sc_edge(h1, p_src, p_dst, rel16, src2, dst2, et2)
    den = _sc_den(p_src, p_dst, rel16, src2, dst2, et2)
    den = den.reshape(NC * HALF, 1)

    latent, h3 = _tc_back(aggp, den, W_lat, W_out,
                          b_out.reshape(1, BASE_DIM), W2,
                          b2.reshape(1, X_DIM))
    return (latent, h3)


# speedup vs baseline: 7.8841x; 1.5944x over previous
import dataclasses

import jax
import jax.numpy as jnp
from jax import lax
from jax.experimental import pallas as pl
from jax.experimental.pallas import tpu as pltpu
from jax.experimental.pallas import tpu_sc as plsc

N_NODES = 10000
N_EDGES = 320000
X_DIM = 128
BASE_DIM = 128
LATENT_DIM = 32

NC = 2
NS = 16
L = 16
GR = 32
EPS = N_EDGES // NS
NCH = 5
NG = EPS // NCH // GR
HALF = 5120
TRASH = HALF
RPS = HALF // NS

_f32 = jnp.float32
_i32 = jnp.int32


def _tc_front_body(f_ref, w1_ref, b1_ref, a2_ref, h1_ref, pv_ref):
    h1 = jnp.maximum(
        jnp.dot(f_ref[...], w1_ref[...], preferred_element_type=_f32)
        + b1_ref[...],
        0.0,
    )
    h1_ref[...] = h1
    pv_ref[...] = jnp.dot(h1, a2_ref[...], preferred_element_type=_f32)


def _tc_front(features, W1, b1, a2):
    BN = 1000
    grid = (N_NODES // BN,)
    return pl.pallas_call(
        _tc_front_body,
        grid=grid,
        in_specs=[
            pl.BlockSpec((BN, X_DIM), lambda i: (i, 0)),
            pl.BlockSpec((X_DIM, BASE_DIM), lambda i: (0, 0)),
            pl.BlockSpec((1, BASE_DIM), lambda i: (0, 0)),
            pl.BlockSpec((BASE_DIM, 8), lambda i: (0, 0)),
        ],
        out_specs=[
            pl.BlockSpec((BN, BASE_DIM), lambda i: (i, 0)),
            pl.BlockSpec((BN, 8), lambda i: (i, 0)),
        ],
        out_shape=[
            jax.ShapeDtypeStruct((N_NODES, BASE_DIM), _f32),
            jax.ShapeDtypeStruct((N_NODES, 8), _f32),
        ],
    )(features, W1, b1, a2)


def _sc_edge_body(h1_hbm, psrc_hbm, pdst_hbm, rel_hbm, src_hbm, dst_hbm,
                  et_hbm, aggp_hbm,
                  psrc_v, pdst_v, rel_v, src2, dst2, et2, x_v, lidx0, lidx1,
                  grows0, grows1, srows0, srows1, gsem0, gsem1, ssem0, ssem1,
                  agg_sh):
    cid = lax.axis_index("c")
    sid = lax.axis_index("s")
    nlo = cid * HALF

    pltpu.sync_copy(psrc_hbm, psrc_v)
    pltpu.sync_copy(pdst_hbm, pdst_v)
    pltpu.sync_copy(rel_hbm, rel_v)

    zeros16 = jnp.zeros((L,), _f32)

    @pl.loop(0, GR)
    def _zero_rows(r):
        for c in range(0, BASE_DIM, L):
            srows0[r, pl.ds(c, L)] = zeros16

    row0 = sid * RPS
    for k in range(RPS // GR):
        pltpu.sync_copy(srows0, agg_sh.at[pl.ds(row0 + k * GR, GR), :])

    @pl.when(sid == 0)
    def _zero_trash():
        pltpu.sync_copy(srows0.at[pl.ds(0, 8), :],
                        agg_sh.at[pl.ds(HALF, 8), :])

    plsc.subcore_barrier()

    def compute_x_lidx(gi, lidx_ref):
        for j in range(0, GR, L):
            sv = src2[gi, pl.ds(j, L)]
            dv = dst2[gi, pl.ds(j, L)]
            tv = et2[gi, pl.ds(j, L)]
            e = (plsc.load_gather(psrc_v, [sv])
                 + plsc.load_gather(pdst_v, [dv])
                 + plsc.load_gather(rel_v, [tv]))
            e = jnp.where(e >= 0.0, e, 0.2 * e)
            x_v[pl.ds(j, L)] = jnp.exp(e)
            lv = dv - nlo
            inb = (lv >= 0) & (lv < HALF)
            lidx_ref[pl.ds(j, L)] = jnp.where(inb, lv, TRASH)

    def scale(grows_ref, srows_ref):
        @pl.loop(0, GR)
        def _scale(r):
            xs = plsc.load_gather(x_v, [jnp.full((L,), r, _i32)])
            for c in range(0, BASE_DIM, L):
                srows_ref[r, pl.ds(c, L)] = grows_ref[r, pl.ds(c, L)] * xs

    def gather_start(gi, grows_ref, sem):
        pltpu.async_copy(h1_hbm.at[src2.at[gi]], grows_ref, sem)

    def gather_wait(gi, grows_ref, sem):
        pltpu.make_async_copy(h1_hbm.at[src2.at[gi]], grows_ref, sem).wait()

    def scatter_start(srows_ref, lidx_ref, sem):
        pltpu.async_copy(srows_ref, agg_sh.at[lidx_ref], sem, add=True)

    def scatter_wait(srows_ref, lidx_ref, sem):
        pltpu.make_async_copy(srows_ref, agg_sh.at[lidx_ref], sem).wait()

    def step(gi, grows_ref, srows_ref, lidx_ref, gsem, ssem,
             wait_scatter, next_gather):
        gather_wait(gi, grows_ref, gsem)
        if wait_scatter is None:
            scatter_wait(srows_ref, lidx_ref, ssem)
        else:
            @pl.when(wait_scatter)
            def _():
                scatter_wait(srows_ref, lidx_ref, ssem)
        compute_x_lidx(gi, lidx_ref)
        scale(grows_ref, srows_ref)
        scatter_start(srows_ref, lidx_ref, ssem)
        if next_gather:
            gather_start(gi + 2, grows_ref, gsem)

    for st in range(NCH):
        plane = sid * NCH + st
        pltpu.sync_copy(src_hbm.at[plane], src2)
        pltpu.sync_copy(dst_hbm.at[plane], dst2)
        pltpu.sync_copy(et_hbm.at[plane], et2)

        gather_start(0, grows0, gsem0)
        gather_start(1, grows1, gsem1)

        @pl.loop(0, (NG - 3) // 2)
        def _pair(k):
            g = 2 * k
            step(g, grows0, srows0, lidx0, gsem0, ssem0, k > 0, True)
            step(g + 1, grows1, srows1, lidx1, gsem1, ssem1, k > 0, True)

        gather_wait(NG - 3, grows0, gsem0)
        scatter_wait(srows0, lidx0, ssem0)
        compute_x_lidx(NG - 3, lidx0)
        scale(grows0, srows0)
        scatter_start(srows0, lidx0, ssem0)
        gather_start(NG - 1, grows0, gsem0)

        step(NG - 2, grows1, srows1, lidx1, gsem1, ssem1, None, False)
        step(NG - 1, grows0, srows0, lidx0, gsem0, ssem0, None, False)

        scatter_wait(srows1, lidx1, ssem1)
        scatter_wait(srows0, lidx0, ssem0)

    plsc.subcore_barrier()

    pltpu.sync_copy(agg_sh.at[pl.ds(row0, RPS)],
                    aggp_hbm.at[cid, pl.ds(row0, RPS), :])


def _sc_edge(h1, p_src, p_dst, rel16, src2, dst2, et2):
    mesh = plsc.VectorSubcoreMesh(core_axis_name="c", subcore_axis_name="s")
    cp = pltpu.CompilerParams()
    if "needs_layout_passes" in pltpu.CompilerParams.__dataclass_fields__:
        cp = dataclasses.replace(cp, needs_layout_passes=False)
    kern = pl.kernel(
        _sc_edge_body,
        out_type=jax.ShapeDtypeStruct((NC, HALF, BASE_DIM), _f32),
        mesh=mesh,
        scratch_types=[
            pltpu.VMEM((N_NODES,), _f32),
            pltpu.VMEM((N_NODES,), _f32),
            pltpu.VMEM((L,), _f32),
            pltpu.VMEM((NG, GR), _i32),
            pltpu.VMEM((NG, GR), _i32),
            pltpu.VMEM((NG, GR), _i32),
            pltpu.VMEM((GR,), _f32),
            pltpu.VMEM((GR,), _i32),
            pltpu.VMEM((GR,), _i32),
            pltpu.VMEM((GR, BASE_DIM), _f32),
            pltpu.VMEM((GR, BASE_DIM), _f32),
            pltpu.VMEM((GR, BASE_DIM), _f32),
            pltpu.VMEM((GR, BASE_DIM), _f32),
            pltpu.SemaphoreType.DMA,
            pltpu.SemaphoreType.DMA,
            pltpu.SemaphoreType.DMA,
            pltpu.SemaphoreType.DMA,
            pltpu.VMEM_SHARED((HALF + 8, BASE_DIM), _f32),
        ],
        compiler_params=cp,
    )
    return kern(h1, p_src, p_dst, rel16, src2, dst2, et2)


NPAD2 = 10240


def _sc_den_body(psrc_hbm, pdst_hbm, rel_hbm, src_hbm, dst_hbm, et_hbm,
                 den_hbm, psrc_v, pdst_v, rel_v, srcd, dstd, etd, x_v, den_v,
                 den_sh):
    cid = lax.axis_index("c")
    sid = lax.axis_index("s")
    row0d = sid * (NPAD2 // NS)

    pltpu.sync_copy(psrc_hbm, psrc_v)
    pltpu.sync_copy(pdst_hbm, pdst_v)
    pltpu.sync_copy(rel_hbm, rel_v)

    zeros16 = jnp.zeros((L,), _f32)
    for j in range(0, GR, L):
        x_v[pl.ds(j, L)] = zeros16
    for k in range((NPAD2 // NS) // GR):
        pltpu.sync_copy(x_v, den_sh.at[pl.ds(row0d + k * GR, GR)])

    plsc.subcore_barrier()

    for st in range(NCH):
        plane = sid * NCH + st
        pltpu.sync_copy(src_hbm.at[plane], srcd)
        pltpu.sync_copy(dst_hbm.at[plane], dstd)
        pltpu.sync_copy(et_hbm.at[plane], etd)

        @pl.loop(0, NG)
        def _granule(gi):
            for j in range(0, GR, L):
                sv = srcd[gi, pl.ds(j, L)]
                dv = dstd[gi, pl.ds(j, L)]
                tv = etd[gi, pl.ds(j, L)]
                e = (plsc.load_gather(psrc_v, [sv])
                     + plsc.load_gather(pdst_v, [dv])
                     + plsc.load_gather(rel_v, [tv]))
                e = jnp.where(e >= 0.0, e, 0.2 * e)
                x_v[pl.ds(j, L)] = jnp.exp(e)
            pltpu.sync_copy(x_v, den_sh.at[dstd.at[gi]], add=True)

    plsc.subcore_barrier()

    pltpu.sync_copy(den_sh.at[pl.ds(cid * HALF + sid * RPS, RPS)], den_v)
    pltpu.sync_copy(den_v, den_hbm.at[cid * NS + sid, 0])


def _sc_den(p_src, p_dst, rel16, src2, dst2, et2):
    mesh = plsc.VectorSubcoreMesh(core_axis_name="c", subcore_axis_name="s")
    cp = pltpu.CompilerParams()
    if "needs_layout_passes" in pltpu.CompilerParams.__dataclass_fields__:
        cp = dataclasses.replace(cp, needs_layout_passes=False)
    kern = pl.kernel(
        _sc_den_body,
        out_type=jax.ShapeDtypeStruct((NC * NS, 1, RPS), _f32),
        mesh=mesh,
        scratch_types=[
            pltpu.VMEM((N_NODES,), _f32),
            pltpu.VMEM((N_NODES,), _f32),
            pltpu.VMEM((L,), _f32),
            pltpu.VMEM((NG, GR), _i32),
            pltpu.VMEM((NG, GR), _i32),
            pltpu.VMEM((NG, GR), _i32),
            pltpu.VMEM((GR,), _f32),
            pltpu.VMEM((RPS,), _f32),
            pltpu.VMEM_SHARED((NPAD2,), _f32),
        ],
        compiler_params=cp,
    )
    return kern(p_src, p_dst, rel16, src2, dst2, et2)


def _tc_back_body(aggp_ref, den_ref, wlat_ref, wout_ref, bout_ref, w2_ref,
                  b2_ref, lat_ref, h3_ref):
    s = jnp.concatenate([aggp_ref[0], aggp_ref[1]], axis=0)
    agg = s / (den_ref[...] + 1e-16)
    lat = jnp.dot(agg, wlat_ref[...], preferred_element_type=_f32)
    lat_ref[...] = lat[:N_NODES]
    h2 = jnp.dot(agg, wout_ref[...], preferred_element_type=_f32) + bout_ref[...]
    h3 = (
        jnp.dot(jnp.maximum(h2, 0.0), w2_ref[...], preferred_element_type=_f32)
        + b2_ref[...]
    )
    h3_ref[...] = h3[:N_NODES]


def _tc_back(aggp, den, W_lat, W_out, b_out, W2, b2):
    return pl.pallas_call(
        _tc_back_body,
        out_shape=[
            jax.ShapeDtypeStruct((N_NODES, LATENT_DIM), _f32),
            jax.ShapeDtypeStruct((N_NODES, X_DIM), _f32),
        ],
    )(aggp, den, W_lat, W_out, b_out, W2, b2)


def kernel(features, edge_index, edge_type, W1, b1, a_src, a_dst, rel_emb,
           W_lat, W_out, b_out, W2, b2):
    a2 = jnp.concatenate(
        [a_src[:, None], a_dst[:, None], jnp.zeros((BASE_DIM, 6), _f32)],
        axis=1,
    )
    h1, pv = _tc_front(features, W1, b1.reshape(1, BASE_DIM), a2)
    p_src = pv[:, 0]
    p_dst = pv[:, 1]

    rel16 = jnp.pad(rel_emb.astype(_f32), (0, L - rel_emb.shape[0]))
    src2 = edge_index[0].reshape(NS * NCH, NG, GR)
    dst2 = edge_index[1].reshape(NS * NCH, NG, GR)
    et2 = edge_type.reshape(NS * NCH, NG, GR)

    aggp = _sc_edge(h1, p_src, p_dst, rel16, src2, dst2, et2)
    den = _sc_den(p_src, p_dst, rel16, src2, dst2, et2)
    den = den.reshape(NC * HALF, 1)

    latent, h3 = _tc_back(aggp, den, W_lat, W_out,
                          b_out.reshape(1, BASE_DIM), W2,
                          b2.reshape(1, X_DIM))
    return (latent, h3)


# speedup vs baseline: 11.3420x; 1.4386x over previous
import dataclasses

import jax
import jax.numpy as jnp
from jax import lax
from jax.experimental import pallas as pl
from jax.experimental.pallas import tpu as pltpu
from jax.experimental.pallas import tpu_sc as plsc

N_NODES = 10000
N_EDGES = 320000
X_DIM = 128
BASE_DIM = 128
LATENT_DIM = 32

NC = 2
NS = 16
L = 16
GR = 32
EPS = N_EDGES // NS
NCH = 5
NG = EPS // NCH // GR
HALF = 5120
TRASH = HALF
RPS = HALF // NS
CH = NG * GR
CLEN = CH + 10 * L

_f32 = jnp.float32
_i32 = jnp.int32


def _tc_front_body(f_ref, w1_ref, b1_ref, a2_ref, h1_ref, pv_ref):
    h1 = jnp.maximum(
        jnp.dot(f_ref[...], w1_ref[...], preferred_element_type=_f32)
        + b1_ref[...],
        0.0,
    )
    h1_ref[...] = h1
    pv_ref[...] = jnp.dot(h1, a2_ref[...], preferred_element_type=_f32)


def _tc_front(features, W1, b1, a2):
    BN = 1000
    grid = (N_NODES // BN,)
    return pl.pallas_call(
        _tc_front_body,
        grid=grid,
        in_specs=[
            pl.BlockSpec((BN, X_DIM), lambda i: (i, 0)),
            pl.BlockSpec((X_DIM, BASE_DIM), lambda i: (0, 0)),
            pl.BlockSpec((1, BASE_DIM), lambda i: (0, 0)),
            pl.BlockSpec((BASE_DIM, 8), lambda i: (0, 0)),
        ],
        out_specs=[
            pl.BlockSpec((BN, BASE_DIM), lambda i: (i, 0)),
            pl.BlockSpec((BN, 8), lambda i: (i, 0)),
        ],
        out_shape=[
            jax.ShapeDtypeStruct((N_NODES, BASE_DIM), _f32),
            jax.ShapeDtypeStruct((N_NODES, 8), _f32),
        ],
    )(features, W1, b1, a2)


def _sc_edge_body(h1_hbm, psrc_hbm, pdst_hbm, rel_hbm, src_hbm, dst_hbm,
                  et_hbm, aggp_hbm,
                  psrc_v, pdst_v, rel_v, src2, dst2, et2,
                  lidx0, lidx1, sidx0, sidx1, grows0, grows1, srows0, srows1,
                  gsem0, gsem1, ssem0, ssem1, agg_sh):
    cid = lax.axis_index("c")
    sid = lax.axis_index("s")
    nlo = cid * HALF

    pltpu.sync_copy(psrc_hbm, psrc_v)
    pltpu.sync_copy(pdst_hbm, pdst_v)
    pltpu.sync_copy(rel_hbm, rel_v)

    zeros16 = jnp.zeros((L,), _f32)

    @pl.loop(0, GR)
    def _zero_rows(r):
        for c in range(0, BASE_DIM, L):
            srows0[r, pl.ds(c, L)] = zeros16

    row0 = sid * RPS
    for k in range(RPS // GR):
        pltpu.sync_copy(srows0, agg_sh.at[pl.ds(row0 + k * GR, GR), :])

    @pl.when(sid == 0)
    def _zero_trash():
        pltpu.sync_copy(srows0.at[pl.ds(0, 8), :],
                        agg_sh.at[pl.ds(HALF, 8), :])

    plsc.subcore_barrier()

    def prep_sidx(gi, sidx_ref):
        gb = gi * GR
        for j in range(0, GR, L):
            sidx_ref[pl.ds(j, L)] = src2[pl.ds(gb + j, L)]

    def prep_lidx(gi, lidx_ref):
        gb = gi * GR
        for j in range(0, GR, L):
            lidx_ref[pl.ds(j, L)] = dst2[pl.ds(gb + j, L)]

    def scale2(grows_ref, srows_ref, gi):
        gb = gi * GR

        @pl.loop(0, GR)
        def _scale(r):
            xs = plsc.bitcast(
                plsc.load_gather(et2, [jnp.full((L,), gb, _i32) + r]), _f32)
            for c in range(0, BASE_DIM, L):
                srows_ref[r, pl.ds(c, L)] = grows_ref[r, pl.ds(c, L)] * xs

    def gather_start(sidx_ref, grows_ref, sem):
        pltpu.async_copy(h1_hbm.at[sidx_ref], grows_ref, sem)

    def gather_wait(sidx_ref, grows_ref, sem):
        pltpu.make_async_copy(h1_hbm.at[sidx_ref], grows_ref, sem).wait()

    def scatter_start(srows_ref, lidx_ref, sem):
        pltpu.async_copy(srows_ref, agg_sh.at[lidx_ref], sem, add=True)

    def scatter_wait(srows_ref, lidx_ref, sem):
        pltpu.make_async_copy(srows_ref, agg_sh.at[lidx_ref], sem).wait()

    def step(gi, grows_ref, srows_ref, sidx_ref, lidx_ref, gsem, ssem,
             wait_scatter, next_gather):
        gather_wait(sidx_ref, grows_ref, gsem)
        if wait_scatter is None:
            scatter_wait(srows_ref, lidx_ref, ssem)
        else:
            @pl.when(wait_scatter)
            def _():
                scatter_wait(srows_ref, lidx_ref, ssem)
        prep_lidx(gi, lidx_ref)
        scale2(grows_ref, srows_ref, gi)
        scatter_start(srows_ref, lidx_ref, ssem)
        if next_gather:
            prep_sidx(gi + 2, sidx_ref)
            gather_start(sidx_ref, grows_ref, gsem)

    trash16 = jnp.full((L,), TRASH, _i32)
    zero16i = jnp.zeros((L,), _i32)

    for st in range(NCH):
        plane = sid * NCH + st
        pltpu.sync_copy(src_hbm.at[plane, 0], src2.at[pl.ds(0, CH)])
        pltpu.sync_copy(dst_hbm.at[plane, 0], dst2.at[pl.ds(0, CH)])
        pltpu.sync_copy(et_hbm.at[plane, 0], et2.at[pl.ds(0, CH)])

        def _compact(g, off):
            j16 = g * L
            sv = src2[pl.ds(j16, L)]
            dv = dst2[pl.ds(j16, L)]
            tv = et2[pl.ds(j16, L)]
            e = (plsc.load_gather(psrc_v, [sv])
                 + plsc.load_gather(pdst_v, [dv])
                 + plsc.load_gather(rel_v, [tv]))
            e = jnp.where(e >= 0.0, e, 0.2 * e)
            xv = jnp.exp(e)
            lv = dv - nlo
            inb = (lv >= 0) & (lv < HALF)
            plsc.store_compressed(src2.at[pl.ds(off, L)], sv, mask=inb)
            plsc.store_compressed(dst2.at[pl.ds(off, L)], lv, mask=inb)
            plsc.store_compressed(et2.at[pl.ds(off, L)],
                                  plsc.bitcast(xv, _i32), mask=inb)
            return off + jnp.sum(inb.astype(_i32), axis=0)

        cnt = lax.fori_loop(0, CH // L, _compact, jnp.int32(0))

        for k in range(9):
            src2[pl.ds(cnt + k * L, L)] = zero16i
            dst2[pl.ds(cnt + k * L, L)] = trash16
            et2[pl.ds(cnt + k * L, L)] = zero16i

        ngr = (cnt + (GR - 1)) // GR
        ngr = jnp.maximum(ngr, 2)
        ngr = ngr | 1

        prep_sidx(0, sidx0)
        gather_start(sidx0, grows0, gsem0)
        prep_sidx(1, sidx1)
        gather_start(sidx1, grows1, gsem1)

        @pl.loop(0, (ngr - 3) // 2)
        def _pair(k):
            g = 2 * k
            step(g, grows0, srows0, sidx0, lidx0, gsem0, ssem0, k > 0, True)
            step(g + 1, grows1, srows1, sidx1, lidx1, gsem1, ssem1, k > 0,
                 True)

        gather_wait(sidx0, grows0, gsem0)
        scatter_wait(srows0, lidx0, ssem0)
        prep_lidx(ngr - 3, lidx0)
        scale2(grows0, srows0, ngr - 3)
        scatter_start(srows0, lidx0, ssem0)
        prep_sidx(ngr - 1, sidx0)
        gather_start(sidx0, grows0, gsem0)

        step(ngr - 2, grows1, srows1, sidx1, lidx1, gsem1, ssem1, None, False)
        step(ngr - 1, grows0, srows0, sidx0, lidx0, gsem0, ssem0, None, False)

        scatter_wait(srows1, lidx1, ssem1)
        scatter_wait(srows0, lidx0, ssem0)

    plsc.subcore_barrier()

    pltpu.sync_copy(agg_sh.at[pl.ds(row0, RPS)],
                    aggp_hbm.at[cid, pl.ds(row0, RPS), :])


def _sc_edge(h1, p_src, p_dst, rel16, src2, dst2, et2):
    mesh = plsc.VectorSubcoreMesh(core_axis_name="c", subcore_axis_name="s")
    cp = pltpu.CompilerParams()
    if "needs_layout_passes" in pltpu.CompilerParams.__dataclass_fields__:
        cp = dataclasses.replace(cp, needs_layout_passes=False)
    kern = pl.kernel(
        _sc_edge_body,
        out_type=jax.ShapeDtypeStruct((NC, HALF, BASE_DIM), _f32),
        mesh=mesh,
        scratch_types=[
            pltpu.VMEM((N_NODES,), _f32),
            pltpu.VMEM((N_NODES,), _f32),
            pltpu.VMEM((L,), _f32),
            pltpu.VMEM((CLEN,), _i32),
            pltpu.VMEM((CLEN,), _i32),
            pltpu.VMEM((CLEN,), _i32),
            pltpu.VMEM((GR,), _i32),
            pltpu.VMEM((GR,), _i32),
            pltpu.VMEM((GR,), _i32),
            pltpu.VMEM((GR,), _i32),
            pltpu.VMEM((GR, BASE_DIM), _f32),
            pltpu.VMEM((GR, BASE_DIM), _f32),
            pltpu.VMEM((GR, BASE_DIM), _f32),
            pltpu.VMEM((GR, BASE_DIM), _f32),
            pltpu.SemaphoreType.DMA,
            pltpu.SemaphoreType.DMA,
            pltpu.SemaphoreType.DMA,
            pltpu.SemaphoreType.DMA,
            pltpu.VMEM_SHARED((HALF + 8, BASE_DIM), _f32),
        ],
        compiler_params=cp,
    )
    return kern(h1, p_src, p_dst, rel16, src2, dst2, et2)


NPAD2 = 10240


def _sc_den_body(psrc_hbm, pdst_hbm, rel_hbm, src_hbm, dst_hbm, et_hbm,
                 den_hbm, psrc_v, pdst_v, rel_v, srcd, dstd, etd, x_v, didx,
                 den_v, den_sh):
    cid = lax.axis_index("c")
    sid = lax.axis_index("s")
    row0d = sid * (NPAD2 // NS)

    pltpu.sync_copy(psrc_hbm, psrc_v)
    pltpu.sync_copy(pdst_hbm, pdst_v)
    pltpu.sync_copy(rel_hbm, rel_v)

    zeros16 = jnp.zeros((L,), _f32)
    for j in range(0, GR, L):
        x_v[pl.ds(j, L)] = zeros16
    for k in range((NPAD2 // NS) // GR):
        pltpu.sync_copy(x_v, den_sh.at[pl.ds(row0d + k * GR, GR)])

    plsc.subcore_barrier()

    for st in range(NCH):
        plane = sid * NCH + st
        pltpu.sync_copy(src_hbm.at[plane, 0], srcd)
        pltpu.sync_copy(dst_hbm.at[plane, 0], dstd)
        pltpu.sync_copy(et_hbm.at[plane, 0], etd)

        @pl.loop(0, NG)
        def _granule(gi):
            gb = gi * GR
            for j in range(0, GR, L):
                sv = srcd[pl.ds(gb + j, L)]
                dv = dstd[pl.ds(gb + j, L)]
                tv = etd[pl.ds(gb + j, L)]
                e = (plsc.load_gather(psrc_v, [sv])
                     + plsc.load_gather(pdst_v, [dv])
                     + plsc.load_gather(rel_v, [tv]))
                e = jnp.where(e >= 0.0, e, 0.2 * e)
                x_v[pl.ds(j, L)] = jnp.exp(e)
                didx[pl.ds(j, L)] = dv
            pltpu.sync_copy(x_v, den_sh.at[didx], add=True)

    plsc.subcore_barrier()

    pltpu.sync_copy(den_sh.at[pl.ds(cid * HALF + sid * RPS, RPS)], den_v)
    pltpu.sync_copy(den_v, den_hbm.at[cid * NS + sid, 0])


def _sc_den(p_src, p_dst, rel16, src2, dst2, et2):
    mesh = plsc.VectorSubcoreMesh(core_axis_name="c", subcore_axis_name="s")
    cp = pltpu.CompilerParams()
    if "needs_layout_passes" in pltpu.CompilerParams.__dataclass_fields__:
        cp = dataclasses.replace(cp, needs_layout_passes=False)
    kern = pl.kernel(
        _sc_den_body,
        out_type=jax.ShapeDtypeStruct((NC * NS, 1, RPS), _f32),
        mesh=mesh,
        scratch_types=[
            pltpu.VMEM((N_NODES,), _f32),
            pltpu.VMEM((N_NODES,), _f32),
            pltpu.VMEM((L,), _f32),
            pltpu.VMEM((CH,), _i32),
            pltpu.VMEM((CH,), _i32),
            pltpu.VMEM((CH,), _i32),
            pltpu.VMEM((GR,), _f32),
            pltpu.VMEM((GR,), _i32),
            pltpu.VMEM((RPS,), _f32),
            pltpu.VMEM_SHARED((NPAD2,), _f32),
        ],
        compiler_params=cp,
    )
    return kern(p_src, p_dst, rel16, src2, dst2, et2)


def _tc_back_body(aggp_ref, den_ref, wlat_ref, wout_ref, bout_ref, w2_ref,
                  b2_ref, lat_ref, h3_ref):
    s = jnp.concatenate([aggp_ref[0], aggp_ref[1]], axis=0)
    agg = s / (den_ref[...] + 1e-16)
    lat = jnp.dot(agg, wlat_ref[...], preferred_element_type=_f32)
    lat_ref[...] = lat[:N_NODES]
    h2 = jnp.dot(agg, wout_ref[...], preferred_element_type=_f32) + bout_ref[...]
    h3 = (
        jnp.dot(jnp.maximum(h2, 0.0), w2_ref[...], preferred_element_type=_f32)
        + b2_ref[...]
    )
    h3_ref[...] = h3[:N_NODES]


def _tc_back(aggp, den, W_lat, W_out, b_out, W2, b2):
    return pl.pallas_call(
        _tc_back_body,
        out_shape=[
            jax.ShapeDtypeStruct((N_NODES, LATENT_DIM), _f32),
            jax.ShapeDtypeStruct((N_NODES, X_DIM), _f32),
        ],
    )(aggp, den, W_lat, W_out, b_out, W2, b2)


def kernel(features, edge_index, edge_type, W1, b1, a_src, a_dst, rel_emb,
           W_lat, W_out, b_out, W2, b2):
    a2 = jnp.concatenate(
        [a_src[:, None], a_dst[:, None], jnp.zeros((BASE_DIM, 6), _f32)],
        axis=1,
    )
    h1, pv = _tc_front(features, W1, b1.reshape(1, BASE_DIM), a2)
    p_src = pv[:, 0]
    p_dst = pv[:, 1]

    rel16 = jnp.pad(rel_emb.astype(_f32), (0, L - rel_emb.shape[0]))
    src2 = edge_index[0].reshape(NS * NCH, 1, CH)
    dst2 = edge_index[1].reshape(NS * NCH, 1, CH)
    et2 = edge_type.reshape(NS * NCH, 1, CH)

    aggp = _sc_edge(h1, p_src, p_dst, rel16, src2, dst2, et2)
    den = _sc_den(p_src, p_dst, rel16, src2, dst2, et2)
    den = den.reshape(NC * HALF, 1)

    latent, h3 = _tc_back(aggp, den, W_lat, W_out,
                          b_out.reshape(1, BASE_DIM), W2,
                          b2.reshape(1, X_DIM))
    return (latent, h3)


# speedup vs baseline: 11.9552x; 1.0541x over previous
import dataclasses

import jax
import jax.numpy as jnp
from jax import lax
from jax.experimental import pallas as pl
from jax.experimental.pallas import tpu as pltpu
from jax.experimental.pallas import tpu_sc as plsc

N_NODES = 10000
N_EDGES = 320000
X_DIM = 128
BASE_DIM = 128
LATENT_DIM = 32

NC = 2
NS = 16
L = 16
GR = 32
EPS = N_EDGES // NS
NCH = 5
NG = EPS // NCH // GR
HALF = 5120
TRASH = HALF
RPS = HALF // NS
CH = NG * GR
CLEN = CH + 10 * L

_f32 = jnp.float32
_i32 = jnp.int32


def _tc_front_body(f_ref, w1_ref, b1_ref, a2_ref, h1_ref, pv_ref):
    h1 = jnp.maximum(
        jnp.dot(f_ref[...], w1_ref[...], preferred_element_type=_f32)
        + b1_ref[...],
        0.0,
    )
    h1_ref[...] = h1
    pv_ref[...] = jnp.dot(h1, a2_ref[...], preferred_element_type=_f32)


def _tc_front(features, W1, b1, a2):
    BN = 1000
    grid = (N_NODES // BN,)
    return pl.pallas_call(
        _tc_front_body,
        grid=grid,
        in_specs=[
            pl.BlockSpec((BN, X_DIM), lambda i: (i, 0)),
            pl.BlockSpec((X_DIM, BASE_DIM), lambda i: (0, 0)),
            pl.BlockSpec((1, BASE_DIM), lambda i: (0, 0)),
            pl.BlockSpec((BASE_DIM, 8), lambda i: (0, 0)),
        ],
        out_specs=[
            pl.BlockSpec((BN, BASE_DIM), lambda i: (i, 0)),
            pl.BlockSpec((BN, 8), lambda i: (i, 0)),
        ],
        out_shape=[
            jax.ShapeDtypeStruct((N_NODES, BASE_DIM), _f32),
            jax.ShapeDtypeStruct((N_NODES, 8), _f32),
        ],
    )(features, W1, b1, a2)


def _sc_edge_body(h1_hbm, psrc_hbm, pdst_hbm, rel_hbm, src_hbm, dst_hbm,
                  et_hbm, aggp_hbm,
                  psrc_v, pdst_v, rel_v, src2, dst2, et2,
                  lidx0, lidx1, sidx0, sidx1, grows0, grows1, srows0, srows1,
                  gsem0, gsem1, ssem0, ssem1, agg_sh):
    cid = lax.axis_index("c")
    sid = lax.axis_index("s")
    nlo = cid * HALF

    pltpu.sync_copy(psrc_hbm, psrc_v)
    pltpu.sync_copy(pdst_hbm, pdst_v)
    pltpu.sync_copy(rel_hbm, rel_v)

    zeros16 = jnp.zeros((L,), _f32)

    @pl.loop(0, GR)
    def _zero_rows(r):
        for c in range(0, BASE_DIM, L):
            srows0[r, pl.ds(c, L)] = zeros16

    row0 = sid * RPS
    for k in range(RPS // GR):
        pltpu.sync_copy(srows0, agg_sh.at[pl.ds(row0 + k * GR, GR), :])

    @pl.when(sid == 0)
    def _zero_trash():
        pltpu.sync_copy(srows0.at[pl.ds(0, 8), :],
                        agg_sh.at[pl.ds(HALF, 8), :])

    plsc.subcore_barrier()

    def prep_sidx(gi, sidx_ref):
        gb = gi * GR
        for j in range(0, GR, L):
            sidx_ref[pl.ds(j, L)] = src2[pl.ds(gb + j, L)]

    def prep_lidx(gi, lidx_ref):
        gb = gi * GR
        for j in range(0, GR, L):
            lidx_ref[pl.ds(j, L)] = dst2[pl.ds(gb + j, L)]

    def scale2(grows_ref, srows_ref, gi):
        gb = gi * GR

        @pl.loop(0, GR)
        def _scale(r):
            xs = plsc.bitcast(
                plsc.load_gather(et2, [jnp.full((L,), gb, _i32) + r]), _f32)
            for c in range(0, BASE_DIM, L):
                srows_ref[r, pl.ds(c, L)] = grows_ref[r, pl.ds(c, L)] * xs

    def gather_start(sidx_ref, grows_ref, sem):
        pltpu.async_copy(h1_hbm.at[sidx_ref], grows_ref, sem)

    def gather_wait(sidx_ref, grows_ref, sem):
        pltpu.make_async_copy(h1_hbm.at[sidx_ref], grows_ref, sem).wait()

    def scatter_start(srows_ref, lidx_ref, sem):
        pltpu.async_copy(srows_ref, agg_sh.at[lidx_ref], sem, add=True)

    def scatter_wait(srows_ref, lidx_ref, sem):
        pltpu.make_async_copy(srows_ref, agg_sh.at[lidx_ref], sem).wait()

    def step(gi, grows_ref, srows_ref, sidx_ref, lidx_ref, gsem, ssem,
             wait_scatter, next_gather):
        gather_wait(sidx_ref, grows_ref, gsem)
        if wait_scatter is None:
            scatter_wait(srows_ref, lidx_ref, ssem)
        else:
            @pl.when(wait_scatter)
            def _():
                scatter_wait(srows_ref, lidx_ref, ssem)
        prep_lidx(gi, lidx_ref)
        scale2(grows_ref, srows_ref, gi)
        scatter_start(srows_ref, lidx_ref, ssem)
        if next_gather:
            prep_sidx(gi + 2, sidx_ref)
            gather_start(sidx_ref, grows_ref, gsem)

    trash16 = jnp.full((L,), TRASH, _i32)
    zero16i = jnp.zeros((L,), _i32)

    for st in range(NCH):
        plane = sid * NCH + st
        pltpu.sync_copy(src_hbm.at[plane, 0], src2.at[pl.ds(0, CH)])
        pltpu.sync_copy(dst_hbm.at[plane, 0], dst2.at[pl.ds(0, CH)])
        pltpu.sync_copy(et_hbm.at[plane, 0], et2.at[pl.ds(0, CH)])

        def _compact(g, off):
            j16 = g * L
            sv = src2[pl.ds(j16, L)]
            dv = dst2[pl.ds(j16, L)]
            tv = et2[pl.ds(j16, L)]
            e = (plsc.load_gather(psrc_v, [sv])
                 + plsc.load_gather(pdst_v, [dv])
                 + plsc.load_gather(rel_v, [tv]))
            e = jnp.where(e >= 0.0, e, 0.2 * e)
            xv = jnp.exp(e)
            lv = dv - nlo
            inb = (lv >= 0) & (lv < HALF)
            plsc.store_compressed(src2.at[pl.ds(off, L)], sv, mask=inb)
            plsc.store_compressed(dst2.at[pl.ds(off, L)], lv, mask=inb)
            plsc.store_compressed(et2.at[pl.ds(off, L)],
                                  plsc.bitcast(xv, _i32), mask=inb)
            return off + jnp.sum(inb.astype(_i32), axis=0)

        cnt = lax.fori_loop(0, CH // L, _compact, jnp.int32(0))

        for k in range(9):
            src2[pl.ds(cnt + k * L, L)] = zero16i
            dst2[pl.ds(cnt + k * L, L)] = trash16
            et2[pl.ds(cnt + k * L, L)] = zero16i

        ngr = (cnt + (GR - 1)) // GR
        ngr = jnp.maximum(ngr, 2)
        ngr = ngr | 1

        prep_sidx(0, sidx0)
        gather_start(sidx0, grows0, gsem0)
        prep_sidx(1, sidx1)
        gather_start(sidx1, grows1, gsem1)

        @pl.loop(0, (ngr - 3) // 2)
        def _pair(k):
            g = 2 * k
            step(g, grows0, srows0, sidx0, lidx0, gsem0, ssem0, k > 0, True)
            step(g + 1, grows1, srows1, sidx1, lidx1, gsem1, ssem1, k > 0,
                 True)

        gather_wait(sidx0, grows0, gsem0)
        scatter_wait(srows0, lidx0, ssem0)
        prep_lidx(ngr - 3, lidx0)
        scale2(grows0, srows0, ngr - 3)
        scatter_start(srows0, lidx0, ssem0)
        prep_sidx(ngr - 1, sidx0)
        gather_start(sidx0, grows0, gsem0)

        step(ngr - 2, grows1, srows1, sidx1, lidx1, gsem1, ssem1, None, False)
        step(ngr - 1, grows0, srows0, sidx0, lidx0, gsem0, ssem0, None, False)

        scatter_wait(srows1, lidx1, ssem1)
        scatter_wait(srows0, lidx0, ssem0)

    plsc.subcore_barrier()

    pltpu.sync_copy(agg_sh.at[pl.ds(row0, RPS)],
                    aggp_hbm.at[cid, pl.ds(row0, RPS), :])


def _sc_edge(h1, p_src, p_dst, rel16, src2, dst2, et2):
    mesh = plsc.VectorSubcoreMesh(core_axis_name="c", subcore_axis_name="s")
    cp = pltpu.CompilerParams()
    if "needs_layout_passes" in pltpu.CompilerParams.__dataclass_fields__:
        cp = dataclasses.replace(cp, needs_layout_passes=False)
    kern = pl.kernel(
        _sc_edge_body,
        out_type=jax.ShapeDtypeStruct((NC, HALF, BASE_DIM), _f32),
        mesh=mesh,
        scratch_types=[
            pltpu.VMEM((N_NODES,), _f32),
            pltpu.VMEM((N_NODES,), _f32),
            pltpu.VMEM((L,), _f32),
            pltpu.VMEM((CLEN,), _i32),
            pltpu.VMEM((CLEN,), _i32),
            pltpu.VMEM((CLEN,), _i32),
            pltpu.VMEM((GR,), _i32),
            pltpu.VMEM((GR,), _i32),
            pltpu.VMEM((GR,), _i32),
            pltpu.VMEM((GR,), _i32),
            pltpu.VMEM((GR, BASE_DIM), _f32),
            pltpu.VMEM((GR, BASE_DIM), _f32),
            pltpu.VMEM((GR, BASE_DIM), _f32),
            pltpu.VMEM((GR, BASE_DIM), _f32),
            pltpu.SemaphoreType.DMA,
            pltpu.SemaphoreType.DMA,
            pltpu.SemaphoreType.DMA,
            pltpu.SemaphoreType.DMA,
            pltpu.VMEM_SHARED((HALF + 8, BASE_DIM), _f32),
        ],
        compiler_params=cp,
    )
    return kern(h1, p_src, p_dst, rel16, src2, dst2, et2)


NPAD2 = 10240


def _sc_den_body(psrc_hbm, pdst_hbm, rel_hbm, src_hbm, dst_hbm, et_hbm,
                 den_hbm, psrc_v, pdst_v, rel_v, srcd, dstd, etd, x_v, x_v1,
                 didx, didx1, dsem0, dsem1, den_v, den_sh):
    cid = lax.axis_index("c")
    sid = lax.axis_index("s")
    row0d = sid * (NPAD2 // NS)

    pltpu.sync_copy(psrc_hbm, psrc_v)
    pltpu.sync_copy(pdst_hbm, pdst_v)
    pltpu.sync_copy(rel_hbm, rel_v)

    zeros16 = jnp.zeros((L,), _f32)
    for j in range(0, GR, L):
        x_v[pl.ds(j, L)] = zeros16
    for k in range((NPAD2 // NS) // GR):
        pltpu.sync_copy(x_v, den_sh.at[pl.ds(row0d + k * GR, GR)])

    plsc.subcore_barrier()

    for st in range(NCH):
        plane = sid * NCH + st
        pltpu.sync_copy(src_hbm.at[plane, 0], srcd)
        pltpu.sync_copy(dst_hbm.at[plane, 0], dstd)
        pltpu.sync_copy(et_hbm.at[plane, 0], etd)

        def dstep(gi, x_ref, d_ref, sem, wait_prev):
            gb = gi * GR
            if wait_prev is None:
                pltpu.make_async_copy(x_ref, den_sh.at[d_ref], sem).wait()
            else:
                @pl.when(wait_prev)
                def _():
                    pltpu.make_async_copy(x_ref, den_sh.at[d_ref],
                                          sem).wait()
            for j in range(0, GR, L):
                sv = srcd[pl.ds(gb + j, L)]
                dv = dstd[pl.ds(gb + j, L)]
                tv = etd[pl.ds(gb + j, L)]
                e = (plsc.load_gather(psrc_v, [sv])
                     + plsc.load_gather(pdst_v, [dv])
                     + plsc.load_gather(rel_v, [tv]))
                e = jnp.where(e >= 0.0, e, 0.2 * e)
                x_ref[pl.ds(j, L)] = jnp.exp(e)
                d_ref[pl.ds(j, L)] = dv
            pltpu.async_copy(x_ref, den_sh.at[d_ref], sem, add=True)

        @pl.loop(0, NG // 2)
        def _dpair(k):
            dstep(2 * k, x_v, didx, dsem0, k > 0)
            dstep(2 * k + 1, x_v1, didx1, dsem1, k > 0)

        dstep(NG - 1, x_v, didx, dsem0, None)
        pltpu.make_async_copy(x_v1, den_sh.at[didx1], dsem1).wait()
        pltpu.make_async_copy(x_v, den_sh.at[didx], dsem0).wait()

    plsc.subcore_barrier()

    pltpu.sync_copy(den_sh.at[pl.ds(cid * HALF + sid * RPS, RPS)], den_v)
    pltpu.sync_copy(den_v, den_hbm.at[cid * NS + sid, 0])


def _sc_den(p_src, p_dst, rel16, src2, dst2, et2):
    mesh = plsc.VectorSubcoreMesh(core_axis_name="c", subcore_axis_name="s")
    cp = pltpu.CompilerParams()
    if "needs_layout_passes" in pltpu.CompilerParams.__dataclass_fields__:
        cp = dataclasses.replace(cp, needs_layout_passes=False)
    kern = pl.kernel(
        _sc_den_body,
        out_type=jax.ShapeDtypeStruct((NC * NS, 1, RPS), _f32),
        mesh=mesh,
        scratch_types=[
            pltpu.VMEM((N_NODES,), _f32),
            pltpu.VMEM((N_NODES,), _f32),
            pltpu.VMEM((L,), _f32),
            pltpu.VMEM((CH,), _i32),
            pltpu.VMEM((CH,), _i32),
            pltpu.VMEM((CH,), _i32),
            pltpu.VMEM((GR,), _f32),
            pltpu.VMEM((GR,), _f32),
            pltpu.VMEM((GR,), _i32),
            pltpu.VMEM((GR,), _i32),
            pltpu.SemaphoreType.DMA,
            pltpu.SemaphoreType.DMA,
            pltpu.VMEM((RPS,), _f32),
            pltpu.VMEM_SHARED((NPAD2,), _f32),
        ],
        compiler_params=cp,
    )
    return kern(p_src, p_dst, rel16, src2, dst2, et2)


def _tc_back_body(aggp_ref, den_ref, wlat_ref, wout_ref, bout_ref, w2_ref,
                  b2_ref, lat_ref, h3_ref):
    s = jnp.concatenate([aggp_ref[0], aggp_ref[1]], axis=0)
    agg = s / (den_ref[...] + 1e-16)
    lat = jnp.dot(agg, wlat_ref[...], preferred_element_type=_f32)
    lat_ref[...] = lat[:N_NODES]
    h2 = jnp.dot(agg, wout_ref[...], preferred_element_type=_f32) + bout_ref[...]
    h3 = (
        jnp.dot(jnp.maximum(h2, 0.0), w2_ref[...], preferred_element_type=_f32)
        + b2_ref[...]
    )
    h3_ref[...] = h3[:N_NODES]


def _tc_back(aggp, den, W_lat, W_out, b_out, W2, b2):
    return pl.pallas_call(
        _tc_back_body,
        out_shape=[
            jax.ShapeDtypeStruct((N_NODES, LATENT_DIM), _f32),
            jax.ShapeDtypeStruct((N_NODES, X_DIM), _f32),
        ],
    )(aggp, den, W_lat, W_out, b_out, W2, b2)


def kernel(features, edge_index, edge_type, W1, b1, a_src, a_dst, rel_emb,
           W_lat, W_out, b_out, W2, b2):
    a2 = jnp.concatenate(
        [a_src[:, None], a_dst[:, None], jnp.zeros((BASE_DIM, 6), _f32)],
        axis=1,
    )
    h1, pv = _tc_front(features, W1, b1.reshape(1, BASE_DIM), a2)
    p_src = pv[:, 0]
    p_dst = pv[:, 1]

    rel16 = jnp.pad(rel_emb.astype(_f32), (0, L - rel_emb.shape[0]))
    src2 = edge_index[0].reshape(NS * NCH, 1, CH)
    dst2 = edge_index[1].reshape(NS * NCH, 1, CH)
    et2 = edge_type.reshape(NS * NCH, 1, CH)

    aggp = _sc_edge(h1, p_src, p_dst, rel16, src2, dst2, et2)
    den = _sc_den(p_src, p_dst, rel16, src2, dst2, et2)
    den = den.reshape(NC * HALF, 1)

    latent, h3 = _tc_back(aggp, den, W_lat, W_out,
                          b_out.reshape(1, BASE_DIM), W2,
                          b2.reshape(1, X_DIM))
    return (latent, h3)


# speedup vs baseline: 16.7501x; 1.4011x over previous
import dataclasses

import jax
import jax.numpy as jnp
from jax import lax
from jax.experimental import pallas as pl
from jax.experimental.pallas import tpu as pltpu
from jax.experimental.pallas import tpu_sc as plsc

N_NODES = 10000
N_EDGES = 320000
X_DIM = 128
BASE_DIM = 128
LATENT_DIM = 32

NC = 2
NS = 16
L = 16
GR = 32
EPS = N_EDGES // NS
NCH = 5
NG = EPS // NCH // GR
HALF = 5120
TRASH = HALF
RPS = HALF // NS
CH = NG * GR
CLEN = CH + 10 * L

_f32 = jnp.float32
_i32 = jnp.int32


def _tc_front_body(f_ref, w1_ref, b1_ref, a2_ref, h1_ref, pv_ref):
    h1 = jnp.maximum(
        jnp.dot(f_ref[...], w1_ref[...], preferred_element_type=_f32)
        + b1_ref[...],
        0.0,
    )
    h1_ref[...] = h1
    pv_ref[...] = jnp.dot(h1, a2_ref[...], preferred_element_type=_f32)


def _tc_front(features, W1, b1, a2):
    BN = 1000
    grid = (N_NODES // BN,)
    return pl.pallas_call(
        _tc_front_body,
        grid=grid,
        in_specs=[
            pl.BlockSpec((BN, X_DIM), lambda i: (i, 0)),
            pl.BlockSpec((X_DIM, BASE_DIM), lambda i: (0, 0)),
            pl.BlockSpec((1, BASE_DIM), lambda i: (0, 0)),
            pl.BlockSpec((BASE_DIM, 8), lambda i: (0, 0)),
        ],
        out_specs=[
            pl.BlockSpec((BN, BASE_DIM), lambda i: (i, 0)),
            pl.BlockSpec((BN, 8), lambda i: (i, 0)),
        ],
        out_shape=[
            jax.ShapeDtypeStruct((N_NODES, BASE_DIM), _f32),
            jax.ShapeDtypeStruct((N_NODES, 8), _f32),
        ],
    )(features, W1, b1, a2)


def _sc_edge_body(h1_hbm, psrc_hbm, pdst_hbm, rel_hbm, src_hbm, dst_hbm,
                  et_hbm, aggp_hbm,
                  psrc_v, pdst_v, rel_v, src2, dst2, et2,
                  lidx0, lidx1, sidx0, sidx1, grows0, grows1, srows0, srows1,
                  gsem0, gsem1, ssem0, ssem1, agg_sh):
    cid = lax.axis_index("c")
    sid = lax.axis_index("s")
    nlo = cid * HALF

    pltpu.sync_copy(psrc_hbm, psrc_v)
    pltpu.sync_copy(pdst_hbm, pdst_v)
    pltpu.sync_copy(rel_hbm, rel_v)

    zeros16 = jnp.zeros((L,), _f32)

    @pl.loop(0, GR)
    def _zero_rows(r):
        for c in range(0, BASE_DIM, L):
            srows0[r, pl.ds(c, L)] = zeros16

    row0 = sid * RPS
    for k in range(RPS // GR):
        pltpu.sync_copy(srows0, agg_sh.at[pl.ds(row0 + k * GR, GR), :])

    @pl.when(sid == 0)
    def _zero_trash():
        pltpu.sync_copy(srows0.at[pl.ds(0, 8), :],
                        agg_sh.at[pl.ds(HALF, 8), :])

    plsc.subcore_barrier()

    def prep_sidx(gi, sidx_ref):
        gb = gi * GR
        for j in range(0, GR, L):
            sidx_ref[pl.ds(j, L)] = src2[pl.ds(gb + j, L)]

    def prep_lidx(gi, lidx_ref):
        gb = gi * GR
        for j in range(0, GR, L):
            lidx_ref[pl.ds(j, L)] = dst2[pl.ds(gb + j, L)]

    def scale2(grows_ref, srows_ref, gi):
        gb = gi * GR

        @plsc.parallel_loop(0, GR, unroll=2)
        def _scale(r):
            xs = plsc.bitcast(
                plsc.load_gather(et2, [jnp.full((L,), gb, _i32) + r]), _f32)
            for c in range(0, BASE_DIM, L):
                srows_ref[r, pl.ds(c, L)] = grows_ref[r, pl.ds(c, L)] * xs

    def gather_start(sidx_ref, grows_ref, sem):
        pltpu.async_copy(h1_hbm.at[sidx_ref], grows_ref, sem)

    def gather_wait(sidx_ref, grows_ref, sem):
        pltpu.make_async_copy(h1_hbm.at[sidx_ref], grows_ref, sem).wait()

    def scatter_start(srows_ref, lidx_ref, sem):
        pltpu.async_copy(srows_ref, agg_sh.at[lidx_ref], sem, add=True)

    def scatter_wait(srows_ref, lidx_ref, sem):
        pltpu.make_async_copy(srows_ref, agg_sh.at[lidx_ref], sem).wait()

    def step(gi, grows_ref, srows_ref, sidx_ref, lidx_ref, gsem, ssem,
             wait_scatter, next_gather):
        gather_wait(sidx_ref, grows_ref, gsem)
        if wait_scatter is None:
            scatter_wait(srows_ref, lidx_ref, ssem)
        else:
            @pl.when(wait_scatter)
            def _():
                scatter_wait(srows_ref, lidx_ref, ssem)
        prep_lidx(gi, lidx_ref)
        scale2(grows_ref, srows_ref, gi)
        scatter_start(srows_ref, lidx_ref, ssem)
        if next_gather:
            prep_sidx(gi + 2, sidx_ref)
            gather_start(sidx_ref, grows_ref, gsem)

    trash16 = jnp.full((L,), TRASH, _i32)
    zero16i = jnp.zeros((L,), _i32)

    for st in range(NCH):
        plane = sid * NCH + st
        pltpu.sync_copy(src_hbm.at[plane, 0], src2.at[pl.ds(0, CH)])
        pltpu.sync_copy(dst_hbm.at[plane, 0], dst2.at[pl.ds(0, CH)])
        pltpu.sync_copy(et_hbm.at[plane, 0], et2.at[pl.ds(0, CH)])

        def _compact(g, off):
            j16 = g * L
            sv = src2[pl.ds(j16, L)]
            dv = dst2[pl.ds(j16, L)]
            tv = et2[pl.ds(j16, L)]
            e = (plsc.load_gather(psrc_v, [sv])
                 + plsc.load_gather(pdst_v, [dv])
                 + plsc.load_gather(rel_v, [tv]))
            e = jnp.where(e >= 0.0, e, 0.2 * e)
            xv = jnp.exp(e)
            lv = dv - nlo
            inb = (lv >= 0) & (lv < HALF)
            plsc.store_compressed(src2.at[pl.ds(off, L)], sv, mask=inb)
            plsc.store_compressed(dst2.at[pl.ds(off, L)], lv, mask=inb)
            plsc.store_compressed(et2.at[pl.ds(off, L)],
                                  plsc.bitcast(xv, _i32), mask=inb)
            return off + jnp.sum(inb.astype(_i32), axis=0)

        cnt = lax.fori_loop(0, CH // L, _compact, jnp.int32(0))

        for k in range(9):
            src2[pl.ds(cnt + k * L, L)] = zero16i
            dst2[pl.ds(cnt + k * L, L)] = trash16
            et2[pl.ds(cnt + k * L, L)] = zero16i

        ngr = (cnt + (GR - 1)) // GR
        ngr = jnp.maximum(ngr, 2)
        ngr = ngr | 1

        prep_sidx(0, sidx0)
        gather_start(sidx0, grows0, gsem0)
        prep_sidx(1, sidx1)
        gather_start(sidx1, grows1, gsem1)

        @pl.loop(0, (ngr - 3) // 2)
        def _pair(k):
            g = 2 * k
            step(g, grows0, srows0, sidx0, lidx0, gsem0, ssem0, k > 0, True)
            step(g + 1, grows1, srows1, sidx1, lidx1, gsem1, ssem1, k > 0,
                 True)

        gather_wait(sidx0, grows0, gsem0)
        scatter_wait(srows0, lidx0, ssem0)
        prep_lidx(ngr - 3, lidx0)
        scale2(grows0, srows0, ngr - 3)
        scatter_start(srows0, lidx0, ssem0)
        prep_sidx(ngr - 1, sidx0)
        gather_start(sidx0, grows0, gsem0)

        step(ngr - 2, grows1, srows1, sidx1, lidx1, gsem1, ssem1, None, False)
        step(ngr - 1, grows0, srows0, sidx0, lidx0, gsem0, ssem0, None, False)

        scatter_wait(srows1, lidx1, ssem1)
        scatter_wait(srows0, lidx0, ssem0)

    plsc.subcore_barrier()

    pltpu.sync_copy(agg_sh.at[pl.ds(row0, RPS)],
                    aggp_hbm.at[cid, pl.ds(row0, RPS), :])


def _sc_edge(h1, p_src, p_dst, rel16, src2, dst2, et2):
    mesh = plsc.VectorSubcoreMesh(core_axis_name="c", subcore_axis_name="s")
    cp = pltpu.CompilerParams()
    if "needs_layout_passes" in pltpu.CompilerParams.__dataclass_fields__:
        cp = dataclasses.replace(cp, needs_layout_passes=False)
    kern = pl.kernel(
        _sc_edge_body,
        out_type=jax.ShapeDtypeStruct((NC, HALF, BASE_DIM), _f32),
        mesh=mesh,
        scratch_types=[
            pltpu.VMEM((N_NODES,), _f32),
            pltpu.VMEM((N_NODES,), _f32),
            pltpu.VMEM((L,), _f32),
            pltpu.VMEM((CLEN,), _i32),
            pltpu.VMEM((CLEN,), _i32),
            pltpu.VMEM((CLEN,), _i32),
            pltpu.VMEM((GR,), _i32),
            pltpu.VMEM((GR,), _i32),
            pltpu.VMEM((GR,), _i32),
            pltpu.VMEM((GR,), _i32),
            pltpu.VMEM((GR, BASE_DIM), _f32),
            pltpu.VMEM((GR, BASE_DIM), _f32),
            pltpu.VMEM((GR, BASE_DIM), _f32),
            pltpu.VMEM((GR, BASE_DIM), _f32),
            pltpu.SemaphoreType.DMA,
            pltpu.SemaphoreType.DMA,
            pltpu.SemaphoreType.DMA,
            pltpu.SemaphoreType.DMA,
            pltpu.VMEM_SHARED((HALF + 8, BASE_DIM), _f32),
        ],
        compiler_params=cp,
    )
    return kern(h1, p_src, p_dst, rel16, src2, dst2, et2)


NPAD2 = 10240


def _sc_den_body(psrc_hbm, pdst_hbm, rel_hbm, src_hbm, dst_hbm, et_hbm,
                 den_hbm, psrc_v, pdst_v, rel_v, srcd, dstd, etd, x_v, x_v1,
                 didx, didx1, dsem0, dsem1, den_v, den_sh):
    cid = lax.axis_index("c")
    sid = lax.axis_index("s")
    row0d = sid * (NPAD2 // NS)

    pltpu.sync_copy(psrc_hbm, psrc_v)
    pltpu.sync_copy(pdst_hbm, pdst_v)
    pltpu.sync_copy(rel_hbm, rel_v)

    zeros16 = jnp.zeros((L,), _f32)
    for j in range(0, GR, L):
        x_v[pl.ds(j, L)] = zeros16
    for k in range((NPAD2 // NS) // GR):
        pltpu.sync_copy(x_v, den_sh.at[pl.ds(row0d + k * GR, GR)])

    plsc.subcore_barrier()

    for st in range(NCH):
        plane = sid * NCH + st
        pltpu.sync_copy(src_hbm.at[plane, 0], srcd)
        pltpu.sync_copy(dst_hbm.at[plane, 0], dstd)
        pltpu.sync_copy(et_hbm.at[plane, 0], etd)

        def dstep(gi, x_ref, d_ref, sem, wait_prev):
            gb = gi * GR
            if wait_prev is None:
                pltpu.make_async_copy(x_ref, den_sh.at[d_ref], sem).wait()
            else:
                @pl.when(wait_prev)
                def _():
                    pltpu.make_async_copy(x_ref, den_sh.at[d_ref],
                                          sem).wait()
            for j in range(0, GR, L):
                sv = srcd[pl.ds(gb + j, L)]
                dv = dstd[pl.ds(gb + j, L)]
                tv = etd[pl.ds(gb + j, L)]
                e = (plsc.load_gather(psrc_v, [sv])
                     + plsc.load_gather(pdst_v, [dv])
                     + plsc.load_gather(rel_v, [tv]))
                e = jnp.where(e >= 0.0, e, 0.2 * e)
                x_ref[pl.ds(j, L)] = jnp.exp(e)
                d_ref[pl.ds(j, L)] = dv
            pltpu.async_copy(x_ref, den_sh.at[d_ref], sem, add=True)

        @pl.loop(0, NG // 2)
        def _dpair(k):
            dstep(2 * k, x_v, didx, dsem0, k > 0)
            dstep(2 * k + 1, x_v1, didx1, dsem1, k > 0)

        dstep(NG - 1, x_v, didx, dsem0, None)
        pltpu.make_async_copy(x_v1, den_sh.at[didx1], dsem1).wait()
        pltpu.make_async_copy(x_v, den_sh.at[didx], dsem0).wait()

    plsc.subcore_barrier()

    pltpu.sync_copy(den_sh.at[pl.ds(cid * HALF + sid * RPS, RPS)], den_v)
    pltpu.sync_copy(den_v, den_hbm.at[cid * NS + sid, 0])


def _sc_den(p_src, p_dst, rel16, src2, dst2, et2):
    mesh = plsc.VectorSubcoreMesh(core_axis_name="c", subcore_axis_name="s")
    cp = pltpu.CompilerParams()
    if "needs_layout_passes" in pltpu.CompilerParams.__dataclass_fields__:
        cp = dataclasses.replace(cp, needs_layout_passes=False)
    kern = pl.kernel(
        _sc_den_body,
        out_type=jax.ShapeDtypeStruct((NC * NS, 1, RPS), _f32),
        mesh=mesh,
        scratch_types=[
            pltpu.VMEM((N_NODES,), _f32),
            pltpu.VMEM((N_NODES,), _f32),
            pltpu.VMEM((L,), _f32),
            pltpu.VMEM((CH,), _i32),
            pltpu.VMEM((CH,), _i32),
            pltpu.VMEM((CH,), _i32),
            pltpu.VMEM((GR,), _f32),
            pltpu.VMEM((GR,), _f32),
            pltpu.VMEM((GR,), _i32),
            pltpu.VMEM((GR,), _i32),
            pltpu.SemaphoreType.DMA,
            pltpu.SemaphoreType.DMA,
            pltpu.VMEM((RPS,), _f32),
            pltpu.VMEM_SHARED((NPAD2,), _f32),
        ],
        compiler_params=cp,
    )
    return kern(p_src, p_dst, rel16, src2, dst2, et2)


def _tc_back_body(aggp_ref, den_ref, wlat_ref, wout_ref, bout_ref, w2_ref,
                  b2_ref, lat_ref, h3_ref):
    s = jnp.concatenate([aggp_ref[0], aggp_ref[1]], axis=0)
    agg = s / (den_ref[...] + 1e-16)
    lat = jnp.dot(agg, wlat_ref[...], preferred_element_type=_f32)
    lat_ref[...] = lat[:N_NODES]
    h2 = jnp.dot(agg, wout_ref[...], preferred_element_type=_f32) + bout_ref[...]
    h3 = (
        jnp.dot(jnp.maximum(h2, 0.0), w2_ref[...], preferred_element_type=_f32)
        + b2_ref[...]
    )
    h3_ref[...] = h3[:N_NODES]


def _tc_back(aggp, den, W_lat, W_out, b_out, W2, b2):
    return pl.pallas_call(
        _tc_back_body,
        out_shape=[
            jax.ShapeDtypeStruct((N_NODES, LATENT_DIM), _f32),
            jax.ShapeDtypeStruct((N_NODES, X_DIM), _f32),
        ],
    )(aggp, den, W_lat, W_out, b_out, W2, b2)


def kernel(features, edge_index, edge_type, W1, b1, a_src, a_dst, rel_emb,
           W_lat, W_out, b_out, W2, b2):
    a2 = jnp.concatenate(
        [a_src[:, None], a_dst[:, None], jnp.zeros((BASE_DIM, 6), _f32)],
        axis=1,
    )
    h1, pv = _tc_front(features, W1, b1.reshape(1, BASE_DIM), a2)
    p_src = pv[:, 0]
    p_dst = pv[:, 1]

    rel16 = jnp.pad(rel_emb.astype(_f32), (0, L - rel_emb.shape[0]))
    src2 = edge_index[0].reshape(NS * NCH, 1, CH)
    dst2 = edge_index[1].reshape(NS * NCH, 1, CH)
    et2 = edge_type.reshape(NS * NCH, 1, CH)

    aggp = _sc_edge(h1, p_src, p_dst, rel16, src2, dst2, et2)
    den = _sc_den(p_src, p_dst, rel16, src2, dst2, et2)
    den = den.reshape(NC * HALF, 1)

    latent, h3 = _tc_back(aggp, den, W_lat, W_out,
                          b_out.reshape(1, BASE_DIM), W2,
                          b2.reshape(1, X_DIM))
    return (latent, h3)
